# Initial kernel scaffold; baseline (speedup 1.0000x reference)
import jax
import jax.numpy as jnp
from jax.experimental import pallas as pl


def kernel(x, edge_index, W1, b1, W2, b2, W3, b3):
    raise NotImplementedError("write your pallas kernel here")


# speedup vs baseline: 4.1376x; 4.1376x over previous
import functools

import jax
import jax.numpy as jnp
from jax import lax
from jax.experimental import pallas as pl
from jax.experimental.pallas import tpu as pltpu
from jax.experimental.pallas import tpu_sc as plsc

N = 250000
E = 2000000
CN16 = 64000
NCH16 = 4
CN32 = 44800
NCH32 = 6
BLK = 128
KSUB = 8
EPT = 125952
EPAD = EPT * 16
NIT = EPT // (KSUB * BLK)
FLB = 400
NTC = 2000


def _make_scatter(F, cn, nch):
    mesh = plsc.VectorSubcoreMesh(core_axis_name="c", subcore_axis_name="s")
    acc_rows = cn + 128
    rpt = acc_rows // 16
    erows = EPT // BLK
    nch2 = nch // 2
    full_nb = cn // FLB
    last_nb = (N - (nch - 1) * cn) // FLB
    jmax = -(-full_nb // 16)

    @functools.partial(
        pl.kernel,
        out_type=jax.ShapeDtypeStruct((N, F), jnp.float32),
        mesh=mesh,
        compiler_params=pltpu.CompilerParams(use_tc_tiling_on_sc=False),
        scratch_types=[
            pltpu.VMEM((KSUB, BLK), jnp.int32),
            pltpu.VMEM((KSUB, BLK), jnp.int32),
            pltpu.VMEM((KSUB, BLK, F), jnp.float32),
            pltpu.VMEM_SHARED((acc_rows, F), jnp.float32),
            pltpu.SemaphoreType.DMA,
            pltpu.SemaphoreType.DMA,
        ],
    )
    def k(x_hbm, src_hbm, idx_hbm, zeros_hbm, out_hbm,
          src_v, idx_v, rows_v, acc, gsem, ssem):
        c = lax.axis_index("c")
        s = lax.axis_index("s")

        def chunk_body(p, carry):
            chunk = nch2 * c + p
            pltpu.sync_copy(zeros_hbm.at[pl.ds(s * rpt, rpt)],
                            acc.at[pl.ds(s * rpt, rpt)])
            plsc.subcore_barrier()

            def it_body(i, carry2):
                row0 = s * erows + i * KSUB
                pltpu.sync_copy(src_hbm.at[pl.ds(row0, KSUB)], src_v)
                pltpu.sync_copy(idx_hbm.at[chunk, pl.ds(row0, KSUB)], idx_v)
                gh = [pltpu.async_copy(x_hbm.at[src_v.at[b]], rows_v.at[b], gsem)
                      for b in range(KSUB)]
                for h in gh:
                    h.wait()
                sh = [pltpu.async_copy(rows_v.at[b], acc.at[idx_v.at[b]], ssem,
                                       add=True)
                      for b in range(KSUB)]
                for h in sh:
                    h.wait()
                return carry2

            lax.fori_loop(0, NIT, it_body, 0)
            plsc.subcore_barrier()

            nbt = jnp.where(chunk == nch - 1, last_nb, full_nb)

            def ob(j, carry3):
                blk_id = s + 16 * j

                @pl.when(blk_id < nbt)
                def _():
                    r0 = blk_id * FLB
                    pltpu.sync_copy(acc.at[pl.ds(r0, FLB)],
                                    out_hbm.at[pl.ds(chunk * cn + r0, FLB)])

                return carry3

            lax.fori_loop(0, jmax, ob, 0)
            plsc.subcore_barrier()
            return carry

        lax.fori_loop(0, nch2, chunk_body, 0)

    return k


_scatter16 = _make_scatter(16, CN16, NCH16)
_scatter32 = _make_scatter(32, CN32, NCH32)


def _tc_pre_body(deg_ref, x_ref, dinv_ref, y_ref):
    d = deg_ref[:, 0:1] + 1.0
    dinv = lax.rsqrt(d)
    dinv_ref[...] = dinv
    y_ref[...] = x_ref[...] * dinv


_tc_pre = pl.pallas_call(
    _tc_pre_body,
    grid=(N // NTC,),
    in_specs=[
        pl.BlockSpec((NTC, 16), lambda i: (i, 0)),
        pl.BlockSpec((NTC, 16), lambda i: (i, 0)),
    ],
    out_specs=[
        pl.BlockSpec((NTC, 1), lambda i: (i, 0)),
        pl.BlockSpec((NTC, 16), lambda i: (i, 0)),
    ],
    out_shape=[
        jax.ShapeDtypeStruct((N, 1), jnp.float32),
        jax.ShapeDtypeStruct((N, 16), jnp.float32),
    ],
)


def _make_tc_layer(fin, fout, relu):
    def body(s_ref, y_ref, dinv_ref, w_ref, b_ref, o_ref):
        dinv = dinv_ref[...]
        z = (s_ref[...] + y_ref[...]) * dinv
        h = lax.dot_general(z, w_ref[...], (((1,), (0,)), ((), ())),
                            preferred_element_type=jnp.float32,
                            precision=lax.Precision.HIGHEST) + b_ref[...]
        if relu:
            h = jnp.maximum(h, 0.0) * dinv
        o_ref[...] = h

    return pl.pallas_call(
        body,
        grid=(N // NTC,),
        in_specs=[
            pl.BlockSpec((NTC, fin), lambda i: (i, 0)),
            pl.BlockSpec((NTC, fin), lambda i: (i, 0)),
            pl.BlockSpec((NTC, 1), lambda i: (i, 0)),
            pl.BlockSpec((fin, fout), lambda i: (0, 0)),
            pl.BlockSpec((1, fout), lambda i: (0, 0)),
        ],
        out_specs=pl.BlockSpec((NTC, fout), lambda i: (i, 0)),
        out_shape=jax.ShapeDtypeStruct((N, fout), jnp.float32),
    )


_tc_l1 = _make_tc_layer(16, 32, True)
_tc_l2 = _make_tc_layer(32, 32, True)
_tc_l3 = _make_tc_layer(32, 144, False)


def kernel(x, edge_index, W1, b1, W2, b2, W3, b3):
    src = edge_index[0].astype(jnp.int32)
    dst = edge_index[1].astype(jnp.int32)
    npad = EPAD - E
    srcp = jnp.concatenate([src, jnp.zeros((npad,), jnp.int32)])
    dstp = jnp.concatenate([dst, jnp.full((npad,), 10 * N, jnp.int32)])
    src2d = srcp.reshape(EPAD // BLK, BLK)

    def chunk_local(cn, nch):
        cids = jnp.arange(nch, dtype=jnp.int32)
        idx = jnp.where(
            (dstp[None, :] // cn) == cids[:, None],
            dstp[None, :] - (cids * cn)[:, None],
            cn,
        ).astype(jnp.int32)
        return idx.reshape(nch, EPAD // BLK, BLK)

    idx16 = chunk_local(CN16, NCH16)
    idx32 = chunk_local(CN32, NCH32)
    ones16 = jnp.ones((N, 16), jnp.float32)
    z16 = jnp.zeros((CN16 + 128, 16), jnp.float32)
    z32 = jnp.zeros((CN32 + 128, 32), jnp.float32)

    deg16 = _scatter16(ones16, src2d, idx16, z16)
    x16 = jnp.pad(x, ((0, 0), (0, 16 - x.shape[1])))
    dinv, y0 = _tc_pre(deg16, x16)
    s0 = _scatter16(y0, src2d, idx16, z16)
    W1p = jnp.pad(W1, ((0, 16 - W1.shape[0]), (0, 0)))
    y1 = _tc_l1(s0, y0, dinv, W1p, b1.reshape(1, -1))
    s1 = _scatter32(y1, src2d, idx32, z32)
    y2 = _tc_l2(s1, y1, dinv, W2, b2.reshape(1, -1))
    s2 = _scatter32(y2, src2d, idx32, z32)
    o = _tc_l3(s2, y2, dinv, W3, b3.reshape(1, -1))
    low = 6000 // 12
    out = o.reshape(1, low, low, 12, 12)
    out = jnp.transpose(out, (0, 1, 3, 2, 4))
    return out.reshape(1, 6000, 6000)
